# baseline (speedup 1.0000x reference)
import functools
import jax
import jax.numpy as jnp
from jax import lax
from jax.experimental import pallas as pl
from jax.experimental.pallas import tpu as pltpu
from jax.experimental.pallas import tpu_sc as plsc

B, C, H, W = 1024, 3, 64, 64
PAD = 3
BN = 128
NK = BN // 16
WIN = 7
RC = 12582912.0


def _body(xt, oh_hbm, ow_hbm, out_hbm,
          oh_v, ow_v, colmap_v, src_v, out_v, gsem, ssem0, ssem1):
    wid = lax.axis_index("s") * 2 + lax.axis_index("c")
    bt = lax.rem(wid, 8)
    hq = lax.div(wid, 8)
    b0 = pl.multiple_of(bt * BN, BN)
    h0 = hq * 16
    pltpu.sync_copy(oh_hbm.at[pl.ds(b0, BN)], oh_v)
    pltpu.sync_copy(ow_hbm.at[pl.ds(b0, BN)], ow_v)
    iota = lax.iota(jnp.int32, 16)
    ssems = (ssem0, ssem1)

    def reflect(i):
        i = jnp.where(i < 0, -i, i)
        return jnp.where(i > H - 1, 2 * (H - 1) - i, i)

    oys = [plsc.load_gather(oh_v, [k * 16 + iota]) for k in range(NK)]
    oxs = [plsc.load_gather(ow_v, [k * 16 + iota]) for k in range(NK)]
    bofs = [k * 16 + iota for k in range(NK)]

    def cm_body(w, carry):
        for kp in range(NK // 2):
            lo = reflect(w - PAD + oxs[2 * kp])
            hi = reflect(w - PAD + oxs[2 * kp + 1])
            colmap_v[w * (NK // 2) + kp, :] = lo + hi * 256
        return carry

    lax.fori_loop(0, W, cm_body, 0)

    def row_dma(c, r):
        slot = lax.rem(r, WIN)
        return pltpu.make_async_copy(
            xt.at[c, r, :, pl.ds(b0, BN)],
            src_v.at[pl.ds(slot * W, W)], gsem)

    def c_body(c, carry):
        for j in range(WIN):
            r = h0 - 3 + j

            @pl.when(r >= 0)
            def _():
                row_dma(c, r).start()

        def hp_body(hp, carry2):
            for p2 in (0, 1):
                h = 2 * hp + p2
                hg = h0 + h

                @pl.when(h == 0)
                def _():
                    for j in range(WIN):
                        r = h0 - 3 + j

                        @pl.when(r >= 0)
                        def _():
                            row_dma(c, 0).wait()

                @pl.when((h >= 1) & (hg + 3 <= H - 1))
                def _():
                    row_dma(c, 0).wait()

                rowbases = []
                for k in range(NK):
                    ry = reflect(hg - PAD + oys[k])
                    rowbases.append(lax.rem(ry, WIN) * W)

                for wh in range(8):
                    @pl.when((c * 16 + h >= 1) | (wh >= 2))
                    def _():
                        pltpu.make_async_copy(
                            out_v.at[wh % 2],
                            out_hbm.at[0, 0, pl.ds(0, 8), pl.ds(0, BN)],
                            ssems[wh % 2]).wait()

                    out_p = out_v.at[wh % 2]

                    @plsc.parallel_loop(0, 8, unroll=4)
                    def _w(w):
                        wg = wh * 8 + w
                        for k in range(NK):
                            e = colmap_v[wg * (NK // 2) + k // 2, :]
                            if k % 2 == 0:
                                rx = jnp.bitwise_and(e, 255)
                            else:
                                rx = lax.shift_right_logical(e, 8)
                            i0 = rowbases[k] + rx
                            v = plsc.load_gather(src_v, [i0, bofs[k]])
                            v = jnp.minimum(v, 255.0)
                            v = jnp.maximum(v, 0.0)
                            v = (v + RC) - RC
                            out_p[w, pl.ds(k * 16, 16)] = v

                    pltpu.async_copy(
                        out_p,
                        out_hbm.at[c, hg, pl.ds(wh * 8, 8), pl.ds(b0, BN)],
                        ssems[wh % 2])

                @pl.when((h <= 14) & (hg + 4 <= H - 1))
                def _():
                    row_dma(c, hg + 4).start()
            return carry2

        lax.fori_loop(0, 8, hp_body, 0)
        return carry

    lax.fori_loop(0, C, c_body, 0)
    for p2 in (0, 1):
        pltpu.make_async_copy(out_v.at[p2],
                              out_hbm.at[0, 0, pl.ds(0, 8), pl.ds(0, BN)],
                              ssems[p2]).wait()


@jax.jit
def kernel(x_uint8, offs_h, offs_w):
    xt = jnp.transpose(x_uint8, (1, 2, 3, 0))
    oh = offs_h.reshape(B).astype(jnp.int32)
    ow = offs_w.reshape(B).astype(jnp.int32)
    mesh = plsc.VectorSubcoreMesh(core_axis_name="c", subcore_axis_name="s")
    run = pl.kernel(
        _body,
        mesh=mesh,
        compiler_params=pltpu.CompilerParams(needs_layout_passes=False),
        out_type=jax.ShapeDtypeStruct((C, H, W, B), jnp.float32),
        scratch_types=[
            pltpu.VMEM((BN,), jnp.int32),
            pltpu.VMEM((BN,), jnp.int32),
            pltpu.VMEM((W * NK // 2, 16), jnp.int32),
            pltpu.VMEM((WIN * W, BN), jnp.float32),
            pltpu.VMEM((2, 8, BN), jnp.float32),
            pltpu.SemaphoreType.DMA,
            pltpu.SemaphoreType.DMA,
            pltpu.SemaphoreType.DMA,
        ],
    )
    out_t = run(xt, oh, ow)
    return jnp.transpose(out_t, (3, 0, 1, 2)).astype(x_uint8.dtype)


# speedup vs baseline: 1.4914x; 1.4914x over previous
import functools
import jax
import jax.numpy as jnp
from jax import lax
from jax.experimental import pallas as pl
from jax.experimental.pallas import tpu as pltpu
from jax.experimental.pallas import tpu_sc as plsc

B, C, H, W = 1024, 3, 64, 64
PAD = 3
ROWS = C * H
NW = 32
SPW = B // NW
RC = 12582912.0


def _body(x_hbm, oh_hbm, ow_hbm, out_hbm, oh_v, ow_v, src_v, out_v, rowmap_v,
          gsem0, gsem1, ssem0, ssem1):
    wid = lax.axis_index("s") * 2 + lax.axis_index("c")
    base = wid * SPW
    pltpu.sync_copy(oh_hbm.at[pl.ds(base, SPW)], oh_v)
    pltpu.sync_copy(ow_hbm.at[pl.ds(base, SPW)], ow_v)
    iota = lax.iota(jnp.int32, 16)
    gsems = (gsem0, gsem1)
    ssems = (ssem0, ssem1)

    def reflect(i, n):
        i = jnp.where(i < 0, -i, i)
        return jnp.where(i > n - 1, 2 * (n - 1) - i, i)

    def start_load(s, p):
        pltpu.async_copy(x_hbm.at[pl.ds((base + s) * ROWS, ROWS)],
                         src_v.at[p], gsems[p])

    start_load(0, 0)
    start_load(1, 1)

    def pair_body(so, carry):
        for p in (0, 1):
            s = 2 * so + p
            pltpu.make_async_copy(x_hbm.at[pl.ds(0, ROWS)], src_v.at[p],
                                  gsems[p]).wait()
            @pl.when(so > 0)
            def _():
                pltpu.make_async_copy(out_v.at[p], out_hbm.at[pl.ds(0, ROWS)],
                                      ssems[p]).wait()

            sv = jnp.full((16,), s, jnp.int32)
            oy = plsc.load_gather(oh_v, [sv])
            ox = plsc.load_gather(ow_v, [sv])
            cols = [reflect(iota + (16 * g - PAD) + ox, W) for g in range(4)]
            for grp in range(ROWS // 16):
                ch = (16 * grp) // H
                ybase = (16 * grp) % H
                ry = reflect(iota + (ybase - PAD) + oy, H)
                rowmap_v[pl.ds(16 * grp, 16)] = ch * H + ry

            src_p = src_v.at[p]
            out_p = out_v.at[p]

            @plsc.parallel_loop(0, ROWS, unroll=16)
            def _row(t):
                tvr = jnp.full((16,), t, jnp.int32)
                rowv = plsc.load_gather(rowmap_v, [tvr])
                for g in range(4):
                    v = plsc.load_gather(src_p, [rowv, cols[g]])
                    v = jnp.minimum(v, 255.0)
                    v = jnp.maximum(v, 0.0)
                    v = (v + RC) - RC
                    out_p[t, pl.ds(16 * g, 16)] = v
            pltpu.async_copy(out_p, out_hbm.at[pl.ds((base + s) * ROWS, ROWS)],
                             ssems[p])

            @pl.when(so < SPW // 2 - 1)
            def _():
                start_load(s + 2, p)
        return carry

    lax.fori_loop(0, SPW // 2, pair_body, 0)
    for p in (0, 1):
        pltpu.make_async_copy(out_v.at[p], out_hbm.at[pl.ds(0, ROWS)],
                              ssems[p]).wait()


@jax.jit
def kernel(x_uint8, offs_h, offs_w):
    x2d = x_uint8.reshape(B * ROWS, W)
    oh = offs_h.reshape(B).astype(jnp.int32)
    ow = offs_w.reshape(B).astype(jnp.int32)
    mesh = plsc.VectorSubcoreMesh(core_axis_name="c", subcore_axis_name="s")
    run = pl.kernel(
        _body,
        mesh=mesh,
        compiler_params=pltpu.CompilerParams(needs_layout_passes=False),
        out_type=jax.ShapeDtypeStruct((B * ROWS, W), jnp.float32),
        scratch_types=[
            pltpu.VMEM((SPW,), jnp.int32),
            pltpu.VMEM((SPW,), jnp.int32),
            pltpu.VMEM((2, ROWS, W), jnp.float32),
            pltpu.VMEM((2, ROWS, W), jnp.float32),
            pltpu.VMEM((ROWS,), jnp.int32),
            pltpu.SemaphoreType.DMA,
            pltpu.SemaphoreType.DMA,
            pltpu.SemaphoreType.DMA,
            pltpu.SemaphoreType.DMA,
        ],
    )
    out = run(x2d, oh, ow)
    return out.reshape(B, C, H, W).astype(x_uint8.dtype)
